# initial kernel scaffold (unmeasured)
import jax
import jax.numpy as jnp
from jax import lax
from jax.experimental import pallas as pl
from jax.experimental.pallas import tpu as pltpu


def kernel(
    x,
):
    def body(*refs):
        pass

    out_shape = jax.ShapeDtypeStruct(..., jnp.float32)
    return pl.pallas_call(body, out_shape=out_shape)(...)


# baseline (device time: 313095 ns/iter reference)
import jax
import jax.numpy as jnp
from jax import lax
from jax.experimental import pallas as pl
from jax.experimental.pallas import tpu as pltpu

N_DEV = 4
M = 8192
N = 1024
CHUNK = M // N_DEV


def kernel(x):
    x16 = x.astype(jnp.bfloat16)

    def body(x_ref, out_ref, send_buf, rs_recv,
             rs_send_sems, rs_recv_sems, ag_send_sems, ag_recv_sems):
        my = lax.axis_index("i")
        left = lax.rem(my - 1 + N_DEV, N_DEV)
        right = lax.rem(my + 1, N_DEV)

        def rows(c):
            return pl.ds(c * CHUNK, CHUNK)

        barrier_sem = pltpu.get_barrier_semaphore()
        for nbr in (left, right):
            pl.semaphore_signal(
                barrier_sem, inc=1,
                device_id=(nbr,), device_id_type=pl.DeviceIdType.MESH,
            )
        pl.semaphore_wait(barrier_sem, 2)

        send_buf[:, :] = x_ref[rows(my), :]
        src = send_buf
        for h in range(N_DEV - 1):
            rdma = pltpu.make_async_remote_copy(
                src_ref=src,
                dst_ref=rs_recv.at[h],
                send_sem=rs_send_sems.at[h],
                recv_sem=rs_recv_sems.at[h],
                device_id=(right,),
                device_id_type=pl.DeviceIdType.MESH,
            )
            rdma.start()
            rdma.wait()
            c = lax.rem(my - h - 1 + N_DEV, N_DEV)
            if h < N_DEV - 2:
                rs_recv[h, :, :] = rs_recv[h, :, :] + x_ref[rows(c), :]
                src = rs_recv.at[h]
            else:
                out_ref[rows(c), :] = rs_recv[h, :, :] + x_ref[rows(c), :]

        for h in range(N_DEV - 1):
            s = lax.rem(my + 1 - h + N_DEV, N_DEV)
            rdma = pltpu.make_async_remote_copy(
                src_ref=out_ref.at[rows(s), :],
                dst_ref=out_ref.at[rows(s), :],
                send_sem=ag_send_sems.at[h],
                recv_sem=ag_recv_sems.at[h],
                device_id=(right,),
                device_id_type=pl.DeviceIdType.MESH,
            )
            rdma.start()
            rdma.wait()

    return pl.pallas_call(
        body,
        out_shape=jax.ShapeDtypeStruct((M, N), jnp.bfloat16),
        in_specs=[pl.BlockSpec(memory_space=pltpu.VMEM)],
        out_specs=pl.BlockSpec(memory_space=pltpu.VMEM),
        scratch_shapes=[
            pltpu.VMEM((CHUNK, N), jnp.bfloat16),
            pltpu.VMEM((N_DEV - 1, CHUNK, N), jnp.bfloat16),
            pltpu.SemaphoreType.DMA((N_DEV - 1,)),
            pltpu.SemaphoreType.DMA((N_DEV - 1,)),
            pltpu.SemaphoreType.DMA((N_DEV - 1,)),
            pltpu.SemaphoreType.DMA((N_DEV - 1,)),
        ],
        compiler_params=pltpu.CompilerParams(collective_id=0),
    )(x16)


# device time: 178118 ns/iter; 1.7578x vs baseline; 1.7578x over previous
import jax
import jax.numpy as jnp
from jax import lax
from jax.experimental import pallas as pl
from jax.experimental.pallas import tpu as pltpu

N_DEV = 4
M = 8192
N = 1024
HALF = M // 2
CHUNK = HALF // N_DEV


def kernel(x):
    x16 = x.astype(jnp.bfloat16)

    def body(x_ref, out_ref, rs_recv_a, rs_recv_b,
             rs_send_sems, rs_recv_sems, ag_send_sems, ag_recv_sems):
        my = lax.axis_index("i")
        left = lax.rem(my - 1 + N_DEV, N_DEV)
        right = lax.rem(my + 1, N_DEV)

        def rows_a(c):
            return pl.ds(c * CHUNK, CHUNK)

        def rows_b(c):
            return pl.ds(HALF + c * CHUNK, CHUNK)

        barrier_sem = pltpu.get_barrier_semaphore()
        for nbr in (left, right):
            pl.semaphore_signal(
                barrier_sem, inc=1,
                device_id=(nbr,), device_id_type=pl.DeviceIdType.MESH,
            )
        pl.semaphore_wait(barrier_sem, 2)

        src_a = x_ref.at[rows_a(my), :]
        src_b = x_ref.at[rows_b(my), :]
        for h in range(N_DEV - 1):
            rdma_a = pltpu.make_async_remote_copy(
                src_ref=src_a,
                dst_ref=rs_recv_a.at[h],
                send_sem=rs_send_sems.at[0, h],
                recv_sem=rs_recv_sems.at[0, h],
                device_id=(right,),
                device_id_type=pl.DeviceIdType.MESH,
            )
            rdma_b = pltpu.make_async_remote_copy(
                src_ref=src_b,
                dst_ref=rs_recv_b.at[h],
                send_sem=rs_send_sems.at[1, h],
                recv_sem=rs_recv_sems.at[1, h],
                device_id=(left,),
                device_id_type=pl.DeviceIdType.MESH,
            )
            rdma_a.start()
            rdma_b.start()
            rdma_a.wait()
            rdma_b.wait()
            ca = lax.rem(my - h - 1 + N_DEV, N_DEV)
            cb = lax.rem(my + h + 1, N_DEV)
            if h < N_DEV - 2:
                rs_recv_a[h, :, :] = rs_recv_a[h, :, :] + x_ref[rows_a(ca), :]
                rs_recv_b[h, :, :] = rs_recv_b[h, :, :] + x_ref[rows_b(cb), :]
                src_a = rs_recv_a.at[h]
                src_b = rs_recv_b.at[h]
            else:
                out_ref[rows_a(ca), :] = rs_recv_a[h, :, :] + x_ref[rows_a(ca), :]
                out_ref[rows_b(cb), :] = rs_recv_b[h, :, :] + x_ref[rows_b(cb), :]

        for h in range(N_DEV - 1):
            sa = lax.rem(my + 1 - h + N_DEV, N_DEV)
            sb = lax.rem(my + 3 + h, N_DEV)
            rdma_a = pltpu.make_async_remote_copy(
                src_ref=out_ref.at[rows_a(sa), :],
                dst_ref=out_ref.at[rows_a(sa), :],
                send_sem=ag_send_sems.at[0, h],
                recv_sem=ag_recv_sems.at[0, h],
                device_id=(right,),
                device_id_type=pl.DeviceIdType.MESH,
            )
            rdma_b = pltpu.make_async_remote_copy(
                src_ref=out_ref.at[rows_b(sb), :],
                dst_ref=out_ref.at[rows_b(sb), :],
                send_sem=ag_send_sems.at[1, h],
                recv_sem=ag_recv_sems.at[1, h],
                device_id=(left,),
                device_id_type=pl.DeviceIdType.MESH,
            )
            rdma_a.start()
            rdma_b.start()
            rdma_a.wait()
            rdma_b.wait()

    return pl.pallas_call(
        body,
        out_shape=jax.ShapeDtypeStruct((M, N), jnp.bfloat16),
        in_specs=[pl.BlockSpec(memory_space=pltpu.VMEM)],
        out_specs=pl.BlockSpec(memory_space=pltpu.VMEM),
        scratch_shapes=[
            pltpu.VMEM((N_DEV - 1, CHUNK, N), jnp.bfloat16),
            pltpu.VMEM((N_DEV - 1, CHUNK, N), jnp.bfloat16),
            pltpu.SemaphoreType.DMA((2, N_DEV - 1)),
            pltpu.SemaphoreType.DMA((2, N_DEV - 1)),
            pltpu.SemaphoreType.DMA((2, N_DEV - 1)),
            pltpu.SemaphoreType.DMA((2, N_DEV - 1)),
        ],
        compiler_params=pltpu.CompilerParams(collective_id=0),
    )(x16)


# device time: 169071 ns/iter; 1.8519x vs baseline; 1.0535x over previous
import jax
import jax.numpy as jnp
from jax import lax
from jax.experimental import pallas as pl
from jax.experimental.pallas import tpu as pltpu

N_DEV = 4
M = 8192
N = 1024
HALF = M // 2
CHUNK = HALF // N_DEV
SUBS = 2
SUB = CHUNK // SUBS
HOPS = N_DEV - 1


def kernel(x):
    x16 = x.astype(jnp.bfloat16)

    def body(x_ref, out_ref, rs_recv_a, rs_recv_b,
             rs_send_sems, rs_recv_sems, ag_send_sems, ag_recv_sems):
        my = lax.axis_index("i")
        left = lax.rem(my - 1 + N_DEV, N_DEV)
        right = lax.rem(my + 1, N_DEV)

        def rows(d, c, s):
            return pl.ds(d * HALF + c * CHUNK + s * SUB, SUB)

        def peer(d):
            return right if d == 0 else left

        def rs_chunk(d, h):
            if d == 0:
                return lax.rem(my - h - 1 + N_DEV, N_DEV)
            return lax.rem(my + h + 1, N_DEV)

        def ag_chunk(d, h):
            if d == 0:
                return lax.rem(my + 1 - h + N_DEV, N_DEV)
            return lax.rem(my + 3 + h, N_DEV)

        rs_recv = (rs_recv_a, rs_recv_b)

        def rs_rdma(d, h, s):
            if h == 0:
                src = x_ref.at[rows(d, my, s), :]
            else:
                src = rs_recv[d].at[h - 1, pl.ds(s * SUB, SUB), :]
            return pltpu.make_async_remote_copy(
                src_ref=src,
                dst_ref=rs_recv[d].at[h, pl.ds(s * SUB, SUB), :],
                send_sem=rs_send_sems.at[d, h, s],
                recv_sem=rs_recv_sems.at[d, h, s],
                device_id=(peer(d),),
                device_id_type=pl.DeviceIdType.MESH,
            )

        def ag_rdma(d, h, s):
            r = rows(d, ag_chunk(d, h), s)
            return pltpu.make_async_remote_copy(
                src_ref=out_ref.at[r, :],
                dst_ref=out_ref.at[r, :],
                send_sem=ag_send_sems.at[d, h, s],
                recv_sem=ag_recv_sems.at[d, h, s],
                device_id=(peer(d),),
                device_id_type=pl.DeviceIdType.MESH,
            )

        barrier_sem = pltpu.get_barrier_semaphore()
        for nbr in (left, right):
            pl.semaphore_signal(
                barrier_sem, inc=1,
                device_id=(nbr,), device_id_type=pl.DeviceIdType.MESH,
            )
        pl.semaphore_wait(barrier_sem, 2)

        for s in range(SUBS):
            for d in range(2):
                rs_rdma(d, 0, s).start()

        for h in range(1, HOPS):
            for s in range(SUBS):
                for d in range(2):
                    rs_rdma(d, h - 1, s).wait_recv()
                    c = rs_chunk(d, h - 1)
                    sl = pl.ds(s * SUB, SUB)
                    rs_recv[d][h - 1, sl, :] = (
                        rs_recv[d][h - 1, sl, :] + x_ref[rows(d, c, s), :]
                    )
                    rs_rdma(d, h, s).start()

        for s in range(SUBS):
            for d in range(2):
                rs_rdma(d, HOPS - 1, s).wait_recv()
                c = rs_chunk(d, HOPS - 1)
                out_ref[rows(d, c, s), :] = (
                    rs_recv[d][HOPS - 1, pl.ds(s * SUB, SUB), :]
                    + x_ref[rows(d, c, s), :]
                )
                ag_rdma(d, 0, s).start()

        for h in range(1, HOPS):
            for s in range(SUBS):
                for d in range(2):
                    ag_rdma(d, h - 1, s).wait_recv()
                    ag_rdma(d, h, s).start()

        for s in range(SUBS):
            for d in range(2):
                ag_rdma(d, HOPS - 1, s).wait_recv()
        for h in range(HOPS):
            for s in range(SUBS):
                for d in range(2):
                    rs_rdma(d, h, s).wait_send()
                    ag_rdma(d, h, s).wait_send()

    return pl.pallas_call(
        body,
        out_shape=jax.ShapeDtypeStruct((M, N), jnp.bfloat16),
        in_specs=[pl.BlockSpec(memory_space=pltpu.VMEM)],
        out_specs=pl.BlockSpec(memory_space=pltpu.VMEM),
        scratch_shapes=[
            pltpu.VMEM((HOPS, CHUNK, N), jnp.bfloat16),
            pltpu.VMEM((HOPS, CHUNK, N), jnp.bfloat16),
            pltpu.SemaphoreType.DMA((2, HOPS, SUBS)),
            pltpu.SemaphoreType.DMA((2, HOPS, SUBS)),
            pltpu.SemaphoreType.DMA((2, HOPS, SUBS)),
            pltpu.SemaphoreType.DMA((2, HOPS, SUBS)),
        ],
        compiler_params=pltpu.CompilerParams(collective_id=0),
    )(x16)


# device time: 149691 ns/iter; 2.0916x vs baseline; 1.1295x over previous
import jax
import jax.numpy as jnp
from jax import lax
from jax.experimental import pallas as pl
from jax.experimental.pallas import tpu as pltpu

N_DEV = 4
M = 8192
N = 1024
HALF = M // 2
CHUNK = HALF // N_DEV
SUBS = 2
SUB = CHUNK // SUBS
HOPS = N_DEV - 1


def kernel(x):
    def body(x_ref, out_ref, stage, send0, rs_recv_a, rs_recv_b,
             dma_sems, rs_send_sems, rs_recv_sems, ag_send_sems, ag_recv_sems):
        my = lax.axis_index("i")
        left = lax.rem(my - 1 + N_DEV, N_DEV)
        right = lax.rem(my + 1, N_DEV)

        def rows(d, c, s):
            return pl.ds(d * HALF + c * CHUNK + s * SUB, SUB)

        def peer(d):
            return right if d == 0 else left

        def rs_chunk(d, h):
            if d == 0:
                return lax.rem(my - h - 1 + N_DEV, N_DEV)
            return lax.rem(my + h + 1, N_DEV)

        def ag_chunk(d, h):
            if d == 0:
                return lax.rem(my + 1 - h + N_DEV, N_DEV)
            return lax.rem(my + 3 + h, N_DEV)

        rs_recv = (rs_recv_a, rs_recv_b)

        def fetch(d, c, slot):
            return pltpu.make_async_copy(
                x_ref.at[pl.ds(d * HALF + c * CHUNK, CHUNK), :],
                stage.at[d, slot],
                dma_sems.at[d, slot],
            )

        def rs_rdma(d, h, s):
            if h == 0:
                src = send0.at[d, pl.ds(s * SUB, SUB), :]
            else:
                src = rs_recv[d].at[h - 1, pl.ds(s * SUB, SUB), :]
            return pltpu.make_async_remote_copy(
                src_ref=src,
                dst_ref=rs_recv[d].at[h, pl.ds(s * SUB, SUB), :],
                send_sem=rs_send_sems.at[d, h, s],
                recv_sem=rs_recv_sems.at[d, h, s],
                device_id=(peer(d),),
                device_id_type=pl.DeviceIdType.MESH,
            )

        def ag_rdma(d, h, s):
            r = rows(d, ag_chunk(d, h), s)
            return pltpu.make_async_remote_copy(
                src_ref=out_ref.at[r, :],
                dst_ref=out_ref.at[r, :],
                send_sem=ag_send_sems.at[d, h, s],
                recv_sem=ag_recv_sems.at[d, h, s],
                device_id=(peer(d),),
                device_id_type=pl.DeviceIdType.MESH,
            )

        for d in range(2):
            fetch(d, my, 0).start()
            fetch(d, rs_chunk(d, 0), 1).start()

        barrier_sem = pltpu.get_barrier_semaphore()
        for nbr in (left, right):
            pl.semaphore_signal(
                barrier_sem, inc=1,
                device_id=(nbr,), device_id_type=pl.DeviceIdType.MESH,
            )
        pl.semaphore_wait(barrier_sem, 2)

        for d in range(2):
            fetch(d, my, 0).wait()
            send0[d, :, :] = stage[d, 0].astype(jnp.bfloat16)
            for s in range(SUBS):
                rs_rdma(d, 0, s).start()
            fetch(d, rs_chunk(d, 1), 0).start()

        for h in range(1, HOPS):
            slot = h % 2
            for d in range(2):
                fetch(d, rs_chunk(d, h - 1), slot).wait()
            for s in range(SUBS):
                for d in range(2):
                    rs_rdma(d, h - 1, s).wait_recv()
                    sl = pl.ds(s * SUB, SUB)
                    rs_recv[d][h - 1, sl, :] = (
                        rs_recv[d][h - 1, sl, :]
                        + stage[d, slot, sl, :].astype(jnp.bfloat16)
                    )
                    rs_rdma(d, h, s).start()
            if h == 1:
                for d in range(2):
                    fetch(d, rs_chunk(d, 2), 1).start()

        for d in range(2):
            fetch(d, rs_chunk(d, HOPS - 1), 1).wait()
        for s in range(SUBS):
            for d in range(2):
                rs_rdma(d, HOPS - 1, s).wait_recv()
                c = rs_chunk(d, HOPS - 1)
                sl = pl.ds(s * SUB, SUB)
                out_ref[rows(d, c, s), :] = (
                    rs_recv[d][HOPS - 1, sl, :]
                    + stage[d, 1, sl, :].astype(jnp.bfloat16)
                )
                ag_rdma(d, 0, s).start()

        for h in range(1, HOPS):
            for s in range(SUBS):
                for d in range(2):
                    ag_rdma(d, h - 1, s).wait_recv()
                    ag_rdma(d, h, s).start()

        for s in range(SUBS):
            for d in range(2):
                ag_rdma(d, HOPS - 1, s).wait_recv()
        for h in range(HOPS):
            for s in range(SUBS):
                for d in range(2):
                    rs_rdma(d, h, s).wait_send()
                    ag_rdma(d, h, s).wait_send()

    return pl.pallas_call(
        body,
        out_shape=jax.ShapeDtypeStruct((M, N), jnp.bfloat16),
        in_specs=[pl.BlockSpec(memory_space=pl.ANY)],
        out_specs=pl.BlockSpec(memory_space=pltpu.VMEM),
        scratch_shapes=[
            pltpu.VMEM((2, 2, CHUNK, N), jnp.float32),
            pltpu.VMEM((2, CHUNK, N), jnp.bfloat16),
            pltpu.VMEM((HOPS, CHUNK, N), jnp.bfloat16),
            pltpu.VMEM((HOPS, CHUNK, N), jnp.bfloat16),
            pltpu.SemaphoreType.DMA((2, 2)),
            pltpu.SemaphoreType.DMA((2, HOPS, SUBS)),
            pltpu.SemaphoreType.DMA((2, HOPS, SUBS)),
            pltpu.SemaphoreType.DMA((2, HOPS, SUBS)),
            pltpu.SemaphoreType.DMA((2, HOPS, SUBS)),
        ],
        compiler_params=pltpu.CompilerParams(collective_id=0),
    )(x)
